# Initial kernel scaffold; baseline (speedup 1.0000x reference)
import jax
import jax.numpy as jnp
from jax.experimental import pallas as pl


def kernel(feats, edge_index, W0, W1):
    raise NotImplementedError("write your pallas kernel here")


# speedup vs baseline: 5.0061x; 5.0061x over previous
import functools

import jax
import jax.numpy as jnp
from jax import lax
from jax.experimental import pallas as pl
from jax.experimental.pallas import tpu as pltpu
from jax.experimental.pallas import tpu_sc as plsc

N = 10000
E = 320000
D_IN = 128
HID = 128
CLS = 40
CP = 48

NC, NS, L = 2, 16, 16
NW = NC * NS
NP = 10240
RPT = NP // NS
EPT = E // NW
EB = 80
NCHUNK = EPT // EB

_mesh = plsc.VectorSubcoreMesh(core_axis_name="c", subcore_axis_name="s")
_sc_params = pltpu.CompilerParams(
    needs_layout_passes=False, use_tc_tiling_on_sc=False)
f32 = jnp.float32


@functools.partial(
    pl.kernel,
    out_type=jax.ShapeDtypeStruct((NC, 2, NP), f32),
    mesh=_mesh,
    compiler_params=_sc_params,
    scratch_types=[
        pltpu.VMEM((EPT,), jnp.int32),
        pltpu.VMEM((EPT,), jnp.int32),
        pltpu.VMEM((NP,), f32),
        pltpu.VMEM((NP,), f32),
        pltpu.VMEM((2 * RPT,), f32),
        pltpu.VMEM((2 * RPT,), f32),
        pltpu.VMEM_SHARED((NS, 2, NP), f32),
    ],
)
def _deg_kernel(src_hbm, dst_hbm, out_hbm, si_v, di_v, d0_v, d1_v,
                racc, rtmp, shared):
    c = lax.axis_index("c")
    s = lax.axis_index("s")
    w = c * NS + s
    base = w * EPT
    pltpu.sync_copy(src_hbm.at[pl.ds(base, EPT)], si_v)
    pltpu.sync_copy(dst_hbm.at[pl.ds(base, EPT)], di_v)

    zero16 = jnp.zeros((L,), f32)

    def zero_body(i, _):
        d0_v[pl.ds(i * L, L)] = zero16
        d1_v[pl.ds(i * L, L)] = zero16
        return _

    lax.fori_loop(0, NP // L, zero_body, None)

    ones = jnp.ones((L,), f32)

    def scat_body(i, _):
        plsc.addupdate_scatter(d0_v, [si_v[pl.ds(i * L, L)]], ones)
        plsc.addupdate_scatter(d1_v, [di_v[pl.ds(i * L, L)]], ones)
        return _

    lax.fori_loop(0, EPT // L, scat_body, None)

    pltpu.sync_copy(d0_v, shared.at[s, 0])
    pltpu.sync_copy(d1_v, shared.at[s, 1])
    plsc.subcore_barrier()

    rows = pl.ds(s * RPT, RPT)
    pltpu.sync_copy(shared.at[0, 0, rows], racc.at[pl.ds(0, RPT)])
    pltpu.sync_copy(shared.at[0, 1, rows], racc.at[pl.ds(RPT, RPT)])

    def red_body(j, _):
        pltpu.sync_copy(shared.at[j, 0, rows], rtmp.at[pl.ds(0, RPT)])
        pltpu.sync_copy(shared.at[j, 1, rows], rtmp.at[pl.ds(RPT, RPT)])

        def add_body(v, carry):
            sl = pl.ds(v * L, L)
            racc[sl] = racc[sl] + rtmp[sl]
            return carry

        lax.fori_loop(0, 2 * RPT // L, add_body, None)
        return _

    lax.fori_loop(1, NS, red_body, None)
    pltpu.sync_copy(racc.at[pl.ds(0, RPT)], out_hbm.at[c, 0, rows])
    pltpu.sync_copy(racc.at[pl.ds(RPT, RPT)], out_hbm.at[c, 1, rows])


def _make_agg(D):
    @functools.partial(
        pl.kernel,
        out_type=jax.ShapeDtypeStruct((NC, NP, D), f32),
        mesh=_mesh,
        compiler_params=_sc_params,
        scratch_types=[
            pltpu.VMEM((EB,), jnp.int32),
            pltpu.VMEM((EB,), jnp.int32),
            pltpu.VMEM((EB, D), f32),
            pltpu.VMEM_SHARED((NP, D), f32),
            pltpu.SemaphoreType.DMA,
        ],
    )
    def _agg(x_hbm, src_hbm, dst_hbm, zeros_hbm, out_hbm, si_v, di_v, rows_v,
             acc, sem):
        c = lax.axis_index("c")
        s = lax.axis_index("s")
        w = c * NS + s

        myrows = pl.ds(s * RPT, RPT)
        pltpu.sync_copy(zeros_hbm.at[myrows], acc.at[myrows])
        plsc.subcore_barrier()

        def chunk_body(i, _):
            e0 = w * EPT + i * EB
            pltpu.sync_copy(src_hbm.at[pl.ds(e0, EB)], si_v)
            pltpu.sync_copy(dst_hbm.at[pl.ds(e0, EB)], di_v)
            pltpu.async_copy(x_hbm.at[si_v], rows_v, sem).wait()
            pltpu.sync_copy(rows_v, acc.at[di_v], add=True)
            return _

        lax.fori_loop(0, NCHUNK, chunk_body, None)
        plsc.subcore_barrier()
        pltpu.sync_copy(acc.at[myrows], out_hbm.at[c].at[myrows])

    return _agg


_agg128 = _make_agg(D_IN)
_agg48 = _make_agg(CP)


_BM = 400
_GRID = N // _BM


def _tc1_body(degp, feats, xn, norms):
    d = degp[0] + degp[1]
    no = lax.rsqrt(jnp.maximum(d[0], 1.0))
    ni = lax.rsqrt(jnp.maximum(d[1], 1.0))
    norms[0] = no
    norms[1] = ni
    xn[...] = feats[...] * no


def _tc1(degp4, feats):
    return pl.pallas_call(
        _tc1_body,
        grid=(_GRID,),
        in_specs=[
            pl.BlockSpec((NC, 2, _BM, 1), lambda i: (0, 0, i, 0)),
            pl.BlockSpec((_BM, D_IN), lambda i: (i, 0)),
        ],
        out_specs=[
            pl.BlockSpec((_BM, D_IN), lambda i: (i, 0)),
            pl.BlockSpec((2, _BM, 1), lambda i: (0, i, 0)),
        ],
        out_shape=[
            jax.ShapeDtypeStruct((N, D_IN), f32),
            jax.ShapeDtypeStruct((2, N, 1), f32),
        ],
    )(degp4, feats)


def _tc2_body(p0, p1, norms, w0, w1, z):
    a = p0[...] + p1[...]
    ni = norms[1]
    no = norms[0]
    h = jnp.dot(a, w0[...], preferred_element_type=f32) * ni
    h = jnp.maximum(h, 0.0) * no
    z[...] = jnp.dot(h, w1[...], preferred_element_type=f32)


def _tc2(p0, p1, norms, w0, w1p):
    return pl.pallas_call(
        _tc2_body,
        grid=(_GRID,),
        in_specs=[
            pl.BlockSpec((_BM, D_IN), lambda i: (i, 0)),
            pl.BlockSpec((_BM, D_IN), lambda i: (i, 0)),
            pl.BlockSpec((2, _BM, 1), lambda i: (0, i, 0)),
            pl.BlockSpec((D_IN, HID), lambda i: (0, 0)),
            pl.BlockSpec((HID, CP), lambda i: (0, 0)),
        ],
        out_specs=pl.BlockSpec((_BM, CP), lambda i: (i, 0)),
        out_shape=jax.ShapeDtypeStruct((N, CP), f32),
    )(p0, p1, norms, w0, w1p)


def _tc3_body(q0, q1, norms, out):
    ni = norms[1]
    out[...] = (q0[...] + q1[...])[:, :CLS] * ni


def _tc3(q0, q1, norms):
    return pl.pallas_call(
        _tc3_body,
        grid=(_GRID,),
        in_specs=[
            pl.BlockSpec((_BM, CP), lambda i: (i, 0)),
            pl.BlockSpec((_BM, CP), lambda i: (i, 0)),
            pl.BlockSpec((2, _BM, 1), lambda i: (0, i, 0)),
        ],
        out_specs=pl.BlockSpec((_BM, CLS), lambda i: (i, 0)),
        out_shape=jax.ShapeDtypeStruct((N, CLS), f32),
    )(q0, q1, norms)


@jax.jit
def kernel(feats, edge_index, W0, W1):
    src = edge_index[0]
    dst = edge_index[1]
    w1p = jnp.pad(W1, ((0, 0), (0, CP - CLS)))

    zeros128 = jnp.zeros((NP, D_IN), f32)
    zeros48 = jnp.zeros((NP, CP), f32)

    degp = _deg_kernel(src, dst)
    degp4 = degp.reshape(NC, 2, NP, 1)
    xn, norms = _tc1(degp4, feats)
    agg1 = _agg128(xn, src, dst, zeros128)
    z = _tc2(agg1[0, :N], agg1[1, :N], norms, W0, w1p)
    agg2 = _agg48(z, src, dst, zeros48)
    return _tc3(agg2[0, :N], agg2[1, :N], norms)


# speedup vs baseline: 7.6714x; 1.5324x over previous
import functools

import jax
import jax.numpy as jnp
from jax import lax
from jax.experimental import pallas as pl
from jax.experimental.pallas import tpu as pltpu
from jax.experimental.pallas import tpu_sc as plsc

N = 10000
E = 320000
D_IN = 128
HID = 128
CLS = 40
CP = 48

NC, NS, L = 2, 16, 16
NW = NC * NS
NP = 10240
RPT = NP // NS
EPT = E // NW
EB = 80
NCHUNK = EPT // EB

_mesh = plsc.VectorSubcoreMesh(core_axis_name="c", subcore_axis_name="s")
_sc_params = pltpu.CompilerParams(
    needs_layout_passes=False, use_tc_tiling_on_sc=False)
f32 = jnp.float32


@functools.partial(
    pl.kernel,
    out_type=jax.ShapeDtypeStruct((NC, 2, NP), f32),
    mesh=_mesh,
    compiler_params=_sc_params,
    scratch_types=[
        pltpu.VMEM((EPT,), jnp.int32),
        pltpu.VMEM((EPT,), jnp.int32),
        pltpu.VMEM((NP,), f32),
        pltpu.VMEM((NP,), f32),
        pltpu.VMEM((2 * RPT,), f32),
        pltpu.VMEM((2 * RPT,), f32),
        pltpu.VMEM_SHARED((NS, 2, NP), f32),
    ],
)
def _deg_kernel(src_hbm, dst_hbm, out_hbm, si_v, di_v, d0_v, d1_v,
                racc, rtmp, shared):
    c = lax.axis_index("c")
    s = lax.axis_index("s")
    w = c * NS + s
    base = w * EPT
    pltpu.sync_copy(src_hbm.at[pl.ds(base, EPT)], si_v)
    pltpu.sync_copy(dst_hbm.at[pl.ds(base, EPT)], di_v)

    zero16 = jnp.zeros((L,), f32)

    def zero_body(i, _):
        d0_v[pl.ds(i * L, L)] = zero16
        d1_v[pl.ds(i * L, L)] = zero16
        return _

    lax.fori_loop(0, NP // L, zero_body, None)

    ones = jnp.ones((L,), f32)

    def scat_body(i, _):
        plsc.addupdate_scatter(d0_v, [si_v[pl.ds(i * L, L)]], ones)
        plsc.addupdate_scatter(d1_v, [di_v[pl.ds(i * L, L)]], ones)
        return _

    lax.fori_loop(0, EPT // L, scat_body, None)

    pltpu.sync_copy(d0_v, shared.at[s, 0])
    pltpu.sync_copy(d1_v, shared.at[s, 1])
    plsc.subcore_barrier()

    rows = pl.ds(s * RPT, RPT)
    pltpu.sync_copy(shared.at[0, 0, rows], racc.at[pl.ds(0, RPT)])
    pltpu.sync_copy(shared.at[0, 1, rows], racc.at[pl.ds(RPT, RPT)])

    def red_body(j, _):
        pltpu.sync_copy(shared.at[j, 0, rows], rtmp.at[pl.ds(0, RPT)])
        pltpu.sync_copy(shared.at[j, 1, rows], rtmp.at[pl.ds(RPT, RPT)])

        def add_body(v, carry):
            sl = pl.ds(v * L, L)
            racc[sl] = racc[sl] + rtmp[sl]
            return carry

        lax.fori_loop(0, 2 * RPT // L, add_body, None)
        return _

    lax.fori_loop(1, NS, red_body, None)
    pltpu.sync_copy(racc.at[pl.ds(0, RPT)], out_hbm.at[c, 0, rows])
    pltpu.sync_copy(racc.at[pl.ds(RPT, RPT)], out_hbm.at[c, 1, rows])


NBUF = 5


def _make_agg(D, eb):
    nchunk = EPT // eb

    @functools.partial(
        pl.kernel,
        out_type=jax.ShapeDtypeStruct((NC, NP, D), f32),
        mesh=_mesh,
        compiler_params=_sc_params,
        scratch_types=[
            pltpu.VMEM((NBUF, eb), jnp.int32),
            pltpu.VMEM((NBUF, eb), jnp.int32),
            pltpu.VMEM((NBUF, eb, D), f32),
            pltpu.VMEM_SHARED((NP, D), f32),
        ] + [pltpu.SemaphoreType.DMA] * NBUF,
    )
    def _agg(x_hbm, src_hbm, dst_hbm, zeros_hbm, out_hbm, si2d, di2d, rows_v,
             acc, *sems):
        c = lax.axis_index("c")
        s = lax.axis_index("s")
        w = c * NS + s

        myrows = pl.ds(s * RPT, RPT)
        pltpu.sync_copy(zeros_hbm.at[myrows], acc.at[myrows])
        plsc.subcore_barrier()

        def chunk_body(k, _):
            chunks = pl.ds(k * NBUF, NBUF)
            pltpu.sync_copy(src_hbm.at[w, chunks], si2d)
            pltpu.sync_copy(dst_hbm.at[w, chunks], di2d)
            descs = []
            for b in range(NBUF):
                descs.append(pltpu.async_copy(
                    x_hbm.at[si2d.at[b]], rows_v.at[b], sems[b]))
            for b in range(NBUF):
                descs[b].wait()
                pltpu.sync_copy(rows_v.at[b], acc.at[di2d.at[b]], add=True)
            return _

        lax.fori_loop(0, nchunk // NBUF, chunk_body, None)
        plsc.subcore_barrier()
        pltpu.sync_copy(acc.at[myrows], out_hbm.at[c].at[myrows])

    return _agg


EB128 = 40
EB48 = 80
_agg128 = _make_agg(D_IN, EB128)
_agg48 = _make_agg(CP, EB48)


_BM = 400
_GRID = N // _BM


def _tc1_body(degp, feats, xn, norms):
    d = degp[0] + degp[1]
    no = lax.rsqrt(jnp.maximum(d[0], 1.0))
    ni = lax.rsqrt(jnp.maximum(d[1], 1.0))
    norms[0] = no
    norms[1] = ni
    xn[...] = feats[...] * no


def _tc1(degp4, feats):
    return pl.pallas_call(
        _tc1_body,
        grid=(_GRID,),
        in_specs=[
            pl.BlockSpec((NC, 2, _BM, 1), lambda i: (0, 0, i, 0)),
            pl.BlockSpec((_BM, D_IN), lambda i: (i, 0)),
        ],
        out_specs=[
            pl.BlockSpec((_BM, D_IN), lambda i: (i, 0)),
            pl.BlockSpec((2, _BM, 1), lambda i: (0, i, 0)),
        ],
        out_shape=[
            jax.ShapeDtypeStruct((N, D_IN), f32),
            jax.ShapeDtypeStruct((2, N, 1), f32),
        ],
    )(degp4, feats)


def _tc2_body(p0, p1, norms, w0, w1, z):
    a = p0[...] + p1[...]
    ni = norms[1]
    no = norms[0]
    h = jnp.dot(a, w0[...], preferred_element_type=f32) * ni
    h = jnp.maximum(h, 0.0) * no
    z[...] = jnp.dot(h, w1[...], preferred_element_type=f32)


def _tc2(p0, p1, norms, w0, w1p):
    return pl.pallas_call(
        _tc2_body,
        grid=(_GRID,),
        in_specs=[
            pl.BlockSpec((_BM, D_IN), lambda i: (i, 0)),
            pl.BlockSpec((_BM, D_IN), lambda i: (i, 0)),
            pl.BlockSpec((2, _BM, 1), lambda i: (0, i, 0)),
            pl.BlockSpec((D_IN, HID), lambda i: (0, 0)),
            pl.BlockSpec((HID, CP), lambda i: (0, 0)),
        ],
        out_specs=pl.BlockSpec((_BM, CP), lambda i: (i, 0)),
        out_shape=jax.ShapeDtypeStruct((N, CP), f32),
    )(p0, p1, norms, w0, w1p)


def _tc3_body(q0, q1, norms, out):
    ni = norms[1]
    out[...] = (q0[...] + q1[...])[:, :CLS] * ni


def _tc3(q0, q1, norms):
    return pl.pallas_call(
        _tc3_body,
        grid=(_GRID,),
        in_specs=[
            pl.BlockSpec((_BM, CP), lambda i: (i, 0)),
            pl.BlockSpec((_BM, CP), lambda i: (i, 0)),
            pl.BlockSpec((2, _BM, 1), lambda i: (0, i, 0)),
        ],
        out_specs=pl.BlockSpec((_BM, CLS), lambda i: (i, 0)),
        out_shape=jax.ShapeDtypeStruct((N, CLS), f32),
    )(q0, q1, norms)


@jax.jit
def kernel(feats, edge_index, W0, W1):
    src = edge_index[0]
    dst = edge_index[1]
    src128 = src.reshape(NW, EPT // EB128, EB128)
    dst128 = dst.reshape(NW, EPT // EB128, EB128)
    src48 = src.reshape(NW, EPT // EB48, EB48)
    dst48 = dst.reshape(NW, EPT // EB48, EB48)
    w1p = jnp.pad(W1, ((0, 0), (0, CP - CLS)))

    zeros128 = jnp.zeros((NP, D_IN), f32)
    zeros48 = jnp.zeros((NP, CP), f32)

    degp = _deg_kernel(src, dst)
    degp4 = degp.reshape(NC, 2, NP, 1)
    xn, norms = _tc1(degp4, feats)
    agg1 = _agg128(xn, src128, dst128, zeros128)
    z = _tc2(agg1[0, :N], agg1[1, :N], norms, W0, w1p)
    agg2 = _agg48(z, src48, dst48, zeros48)
    return _tc3(agg2[0, :N], agg2[1, :N], norms)


# speedup vs baseline: 9.7157x; 1.2665x over previous
import functools

import jax
import jax.numpy as jnp
from jax import lax
from jax.experimental import pallas as pl
from jax.experimental.pallas import tpu as pltpu
from jax.experimental.pallas import tpu_sc as plsc

N = 10000
E = 320000
D_IN = 128
HID = 128
CLS = 40
CP = 48

NC, NS, L = 2, 16, 16
NW = NC * NS
NP = 10240
RPT = NP // NS
EPT = E // NW
EB = 80
NCHUNK = EPT // EB

_mesh = plsc.VectorSubcoreMesh(core_axis_name="c", subcore_axis_name="s")
_sc_params = pltpu.CompilerParams(
    needs_layout_passes=False, use_tc_tiling_on_sc=False)
f32 = jnp.float32
bf16 = jnp.bfloat16


@functools.partial(
    pl.kernel,
    out_type=jax.ShapeDtypeStruct((NC, 2, NP), f32),
    mesh=_mesh,
    compiler_params=_sc_params,
    scratch_types=[
        pltpu.VMEM((EPT,), jnp.int32),
        pltpu.VMEM((EPT,), jnp.int32),
        pltpu.VMEM((NP,), f32),
        pltpu.VMEM((NP,), f32),
        pltpu.VMEM((2 * RPT,), f32),
        pltpu.VMEM((2 * RPT,), f32),
        pltpu.VMEM_SHARED((NS, 2, NP), f32),
    ],
)
def _deg_kernel(src_hbm, dst_hbm, out_hbm, si_v, di_v, d0_v, d1_v,
                racc, rtmp, shared):
    c = lax.axis_index("c")
    s = lax.axis_index("s")
    w = c * NS + s
    base = w * EPT
    pltpu.sync_copy(src_hbm.at[pl.ds(base, EPT)], si_v)
    pltpu.sync_copy(dst_hbm.at[pl.ds(base, EPT)], di_v)

    zero16 = jnp.zeros((L,), f32)

    def zero_body(i, _):
        d0_v[pl.ds(i * L, L)] = zero16
        d1_v[pl.ds(i * L, L)] = zero16
        return _

    lax.fori_loop(0, NP // L, zero_body, None)

    ones = jnp.ones((L,), f32)

    def scat_body(i, _):
        plsc.addupdate_scatter(d0_v, [si_v[pl.ds(i * L, L)]], ones)
        plsc.addupdate_scatter(d1_v, [di_v[pl.ds(i * L, L)]], ones)
        return _

    lax.fori_loop(0, EPT // L, scat_body, None)

    pltpu.sync_copy(d0_v, shared.at[s, 0])
    pltpu.sync_copy(d1_v, shared.at[s, 1])
    plsc.subcore_barrier()

    rows = pl.ds(s * RPT, RPT)
    pltpu.sync_copy(shared.at[0, 0, rows], racc.at[pl.ds(0, RPT)])
    pltpu.sync_copy(shared.at[0, 1, rows], racc.at[pl.ds(RPT, RPT)])

    def red_body(j, _):
        pltpu.sync_copy(shared.at[j, 0, rows], rtmp.at[pl.ds(0, RPT)])
        pltpu.sync_copy(shared.at[j, 1, rows], rtmp.at[pl.ds(RPT, RPT)])

        def add_body(v, carry):
            sl = pl.ds(v * L, L)
            racc[sl] = racc[sl] + rtmp[sl]
            return carry

        lax.fori_loop(0, 2 * RPT // L, add_body, None)
        return _

    lax.fori_loop(1, NS, red_body, None)
    pltpu.sync_copy(racc.at[pl.ds(0, RPT)], out_hbm.at[c, 0, rows])
    pltpu.sync_copy(racc.at[pl.ds(RPT, RPT)], out_hbm.at[c, 1, rows])


NBUF = 5


def _make_agg(D, eb, dt):
    nchunk = EPT // eb

    @functools.partial(
        pl.kernel,
        out_type=jax.ShapeDtypeStruct((NC, NP, D), dt),
        mesh=_mesh,
        compiler_params=_sc_params,
        scratch_types=[
            pltpu.VMEM((NBUF, eb), jnp.int32),
            pltpu.VMEM((NBUF, eb), jnp.int32),
            pltpu.VMEM((NBUF, eb, D), dt),
            pltpu.VMEM_SHARED((NP, D), dt),
        ] + [pltpu.SemaphoreType.DMA] * NBUF,
    )
    def _agg(x_hbm, src_hbm, dst_hbm, zeros_hbm, out_hbm, si2d, di2d, rows_v,
             acc, *sems):
        c = lax.axis_index("c")
        s = lax.axis_index("s")
        w = c * NS + s

        myrows = pl.ds(s * RPT, RPT)
        pltpu.sync_copy(zeros_hbm.at[myrows], acc.at[myrows])
        plsc.subcore_barrier()

        def chunk_body(k, _):
            chunks = pl.ds(k * NBUF, NBUF)
            pltpu.sync_copy(src_hbm.at[w, chunks], si2d)
            pltpu.sync_copy(dst_hbm.at[w, chunks], di2d)
            descs = []
            for b in range(NBUF):
                descs.append(pltpu.async_copy(
                    x_hbm.at[si2d.at[b]], rows_v.at[b], sems[b]))
            for b in range(NBUF):
                descs[b].wait()
                pltpu.sync_copy(rows_v.at[b], acc.at[di2d.at[b]], add=True)
            return _

        lax.fori_loop(0, nchunk // NBUF, chunk_body, None)
        plsc.subcore_barrier()
        pltpu.sync_copy(acc.at[myrows], out_hbm.at[c].at[myrows])

    return _agg


EB128 = 80
EB48 = 80
_agg128 = _make_agg(D_IN, EB128, bf16)
_agg48 = _make_agg(CP, EB48, bf16)


_BM = 400
_GRID = N // _BM


def _tc1_body(degp, feats, xn, norms):
    d = degp[0] + degp[1]
    no = lax.rsqrt(jnp.maximum(d[0], 1.0))
    ni = lax.rsqrt(jnp.maximum(d[1], 1.0))
    norms[0] = no
    norms[1] = ni
    xn[...] = (feats[...] * no).astype(bf16)


def _tc1(degp4, feats):
    return pl.pallas_call(
        _tc1_body,
        grid=(_GRID,),
        in_specs=[
            pl.BlockSpec((NC, 2, _BM, 1), lambda i: (0, 0, i, 0)),
            pl.BlockSpec((_BM, D_IN), lambda i: (i, 0)),
        ],
        out_specs=[
            pl.BlockSpec((_BM, D_IN), lambda i: (i, 0)),
            pl.BlockSpec((2, _BM, 1), lambda i: (0, i, 0)),
        ],
        out_shape=[
            jax.ShapeDtypeStruct((N, D_IN), bf16),
            jax.ShapeDtypeStruct((2, N, 1), f32),
        ],
    )(degp4, feats)


def _tc2_body(p0, p1, norms, w0, w1, z):
    a = p0[...].astype(f32) + p1[...].astype(f32)
    ni = norms[1]
    no = norms[0]
    h = jnp.dot(a, w0[...], preferred_element_type=f32) * ni
    h = jnp.maximum(h, 0.0) * no
    z[...] = jnp.dot(h, w1[...], preferred_element_type=f32).astype(bf16)


def _tc2(p0, p1, norms, w0, w1p):
    return pl.pallas_call(
        _tc2_body,
        grid=(_GRID,),
        in_specs=[
            pl.BlockSpec((_BM, D_IN), lambda i: (i, 0)),
            pl.BlockSpec((_BM, D_IN), lambda i: (i, 0)),
            pl.BlockSpec((2, _BM, 1), lambda i: (0, i, 0)),
            pl.BlockSpec((D_IN, HID), lambda i: (0, 0)),
            pl.BlockSpec((HID, CP), lambda i: (0, 0)),
        ],
        out_specs=pl.BlockSpec((_BM, CP), lambda i: (i, 0)),
        out_shape=jax.ShapeDtypeStruct((N, CP), bf16),
    )(p0, p1, norms, w0, w1p)


def _tc3_body(q0, q1, norms, out):
    ni = norms[1]
    out[...] = (q0[...].astype(f32) + q1[...].astype(f32))[:, :CLS] * ni


def _tc3(q0, q1, norms):
    return pl.pallas_call(
        _tc3_body,
        grid=(_GRID,),
        in_specs=[
            pl.BlockSpec((_BM, CP), lambda i: (i, 0)),
            pl.BlockSpec((_BM, CP), lambda i: (i, 0)),
            pl.BlockSpec((2, _BM, 1), lambda i: (0, i, 0)),
        ],
        out_specs=pl.BlockSpec((_BM, CLS), lambda i: (i, 0)),
        out_shape=jax.ShapeDtypeStruct((N, CLS), f32),
    )(q0, q1, norms)


@jax.jit
def kernel(feats, edge_index, W0, W1):
    src = edge_index[0]
    dst = edge_index[1]
    src128 = src.reshape(NW, EPT // EB128, EB128)
    dst128 = dst.reshape(NW, EPT // EB128, EB128)
    src48 = src.reshape(NW, EPT // EB48, EB48)
    dst48 = dst.reshape(NW, EPT // EB48, EB48)
    w1p = jnp.pad(W1, ((0, 0), (0, CP - CLS)))

    zeros128 = jnp.zeros((NP, D_IN), bf16)
    zeros48 = jnp.zeros((NP, CP), bf16)

    degp = _deg_kernel(src, dst)
    degp4 = degp.reshape(NC, 2, NP, 1)
    xn, norms = _tc1(degp4, feats)
    agg1 = _agg128(xn, src128, dst128, zeros128)
    z = _tc2(agg1[0, :N], agg1[1, :N], norms, W0, w1p)
    agg2 = _agg48(z, src48, dst48, zeros48)
    return _tc3(agg2[0, :N], agg2[1, :N], norms)


# speedup vs baseline: 10.2638x; 1.0564x over previous
import functools

import jax
import jax.numpy as jnp
from jax import lax
from jax.experimental import pallas as pl
from jax.experimental.pallas import tpu as pltpu
from jax.experimental.pallas import tpu_sc as plsc

N = 10000
E = 320000
D_IN = 128
HID = 128
CLS = 40
CP = 48

NC, NS, L = 2, 16, 16
NW = NC * NS
NP = 10240
RPT = NP // NS
EPT = E // NW
EB = 80
NCHUNK = EPT // EB

_mesh = plsc.VectorSubcoreMesh(core_axis_name="c", subcore_axis_name="s")
_sc_params = pltpu.CompilerParams(
    needs_layout_passes=False, use_tc_tiling_on_sc=False)
f32 = jnp.float32
bf16 = jnp.bfloat16


@functools.partial(
    pl.kernel,
    out_type=jax.ShapeDtypeStruct((NC, 2, NP), f32),
    mesh=_mesh,
    compiler_params=_sc_params,
    scratch_types=[
        pltpu.VMEM((EPT,), jnp.int32),
        pltpu.VMEM((EPT,), jnp.int32),
        pltpu.VMEM((NP,), f32),
        pltpu.VMEM((NP,), f32),
        pltpu.VMEM((2 * RPT,), f32),
        pltpu.VMEM((2 * RPT,), f32),
        pltpu.VMEM_SHARED((NS, 2, NP), f32),
    ],
)
def _deg_kernel(src_hbm, dst_hbm, out_hbm, si_v, di_v, d0_v, d1_v,
                racc, rtmp, shared):
    c = lax.axis_index("c")
    s = lax.axis_index("s")
    w = c * NS + s
    base = w * EPT
    pltpu.sync_copy(src_hbm.at[pl.ds(base, EPT)], si_v)
    pltpu.sync_copy(dst_hbm.at[pl.ds(base, EPT)], di_v)

    zero16 = jnp.zeros((L,), f32)

    def zero_body(i, _):
        d0_v[pl.ds(i * L, L)] = zero16
        d1_v[pl.ds(i * L, L)] = zero16
        return _

    lax.fori_loop(0, NP // L, zero_body, None)

    ones = jnp.ones((L,), f32)

    def scat_body(i, _):
        plsc.addupdate_scatter(d0_v, [si_v[pl.ds(i * L, L)]], ones)
        plsc.addupdate_scatter(d1_v, [di_v[pl.ds(i * L, L)]], ones)
        return _

    lax.fori_loop(0, EPT // L, scat_body, None)

    pltpu.sync_copy(d0_v, shared.at[s, 0])
    pltpu.sync_copy(d1_v, shared.at[s, 1])
    plsc.subcore_barrier()

    rows = pl.ds(s * RPT, RPT)
    pltpu.sync_copy(shared.at[0, 0, rows], racc.at[pl.ds(0, RPT)])
    pltpu.sync_copy(shared.at[0, 1, rows], racc.at[pl.ds(RPT, RPT)])

    def red_body(j, _):
        pltpu.sync_copy(shared.at[j, 0, rows], rtmp.at[pl.ds(0, RPT)])
        pltpu.sync_copy(shared.at[j, 1, rows], rtmp.at[pl.ds(RPT, RPT)])

        def add_body(v, carry):
            sl = pl.ds(v * L, L)
            racc[sl] = racc[sl] + rtmp[sl]
            return carry

        lax.fori_loop(0, 2 * RPT // L, add_body, None)
        return _

    lax.fori_loop(1, NS, red_body, None)
    pltpu.sync_copy(racc.at[pl.ds(0, RPT)], out_hbm.at[c, 0, rows])
    pltpu.sync_copy(racc.at[pl.ds(RPT, RPT)], out_hbm.at[c, 1, rows])


NBUF = 5


def _make_agg(D, eb, dt):
    nchunk = EPT // eb

    @functools.partial(
        pl.kernel,
        out_type=jax.ShapeDtypeStruct((NC, NP, D), dt),
        mesh=_mesh,
        compiler_params=_sc_params,
        scratch_types=[
            pltpu.VMEM((NBUF, eb), jnp.int32),
            pltpu.VMEM((NBUF, eb), jnp.int32),
            pltpu.VMEM((NBUF, eb, D), dt),
            pltpu.VMEM_SHARED((NP, D), dt),
            pltpu.VMEM_SHARED((NP, D), dt),
        ] + [pltpu.SemaphoreType.DMA] * NBUF,
    )
    def _agg(x_hbm, src_hbm, dst_hbm, zeros_hbm, out_hbm, si2d, di2d, rows_v,
             acc, xtab, *sems):
        c = lax.axis_index("c")
        s = lax.axis_index("s")
        w = c * NS + s

        myrows = pl.ds(s * RPT, RPT)
        pltpu.sync_copy(zeros_hbm.at[myrows], acc.at[myrows])
        pltpu.sync_copy(x_hbm.at[myrows], xtab.at[myrows])
        plsc.subcore_barrier()

        def chunk_body(k, _):
            chunks = pl.ds(k * NBUF, NBUF)
            pltpu.sync_copy(src_hbm.at[w, chunks], si2d)
            pltpu.sync_copy(dst_hbm.at[w, chunks], di2d)
            descs = []
            for b in range(NBUF):
                descs.append(pltpu.async_copy(
                    xtab.at[si2d.at[b]], rows_v.at[b], sems[b]))
            for b in range(NBUF):
                descs[b].wait()
                pltpu.sync_copy(rows_v.at[b], acc.at[di2d.at[b]], add=True)
            return _

        lax.fori_loop(0, nchunk // NBUF, chunk_body, None)
        plsc.subcore_barrier()
        pltpu.sync_copy(acc.at[myrows], out_hbm.at[c].at[myrows])

    return _agg


EB128 = 80
EB48 = 80
_agg128 = _make_agg(D_IN, EB128, bf16)
_agg48 = _make_agg(CP, EB48, bf16)


_BM = 400
_GRID = N // _BM


def _tc1_body(degp, feats, xn, norms):
    d = degp[0] + degp[1]
    no = lax.rsqrt(jnp.maximum(d[0], 1.0))
    ni = lax.rsqrt(jnp.maximum(d[1], 1.0))
    norms[0] = no
    norms[1] = ni
    xn[...] = (feats[...] * no).astype(bf16)


def _tc1(degp4, feats):
    return pl.pallas_call(
        _tc1_body,
        grid=(_GRID,),
        in_specs=[
            pl.BlockSpec((NC, 2, _BM, 1), lambda i: (0, 0, i, 0)),
            pl.BlockSpec((_BM, D_IN), lambda i: (i, 0)),
        ],
        out_specs=[
            pl.BlockSpec((_BM, D_IN), lambda i: (i, 0)),
            pl.BlockSpec((2, _BM, 1), lambda i: (0, i, 0)),
        ],
        out_shape=[
            jax.ShapeDtypeStruct((N, D_IN), bf16),
            jax.ShapeDtypeStruct((2, N, 1), f32),
        ],
    )(degp4, feats)


def _tc2_body(p0, p1, norms, w0, w1, z):
    a = p0[...].astype(f32) + p1[...].astype(f32)
    ni = norms[1]
    no = norms[0]
    h = jnp.dot(a, w0[...], preferred_element_type=f32) * ni
    h = jnp.maximum(h, 0.0) * no
    z[...] = jnp.dot(h, w1[...], preferred_element_type=f32).astype(bf16)


def _tc2(p0, p1, norms, w0, w1p):
    return pl.pallas_call(
        _tc2_body,
        grid=(_GRID,),
        in_specs=[
            pl.BlockSpec((_BM, D_IN), lambda i: (i, 0)),
            pl.BlockSpec((_BM, D_IN), lambda i: (i, 0)),
            pl.BlockSpec((2, _BM, 1), lambda i: (0, i, 0)),
            pl.BlockSpec((D_IN, HID), lambda i: (0, 0)),
            pl.BlockSpec((HID, CP), lambda i: (0, 0)),
        ],
        out_specs=pl.BlockSpec((_BM, CP), lambda i: (i, 0)),
        out_shape=jax.ShapeDtypeStruct((N, CP), bf16),
    )(p0, p1, norms, w0, w1p)


def _tc3_body(q0, q1, norms, out):
    ni = norms[1]
    out[...] = (q0[...].astype(f32) + q1[...].astype(f32))[:, :CLS] * ni


def _tc3(q0, q1, norms):
    return pl.pallas_call(
        _tc3_body,
        grid=(_GRID,),
        in_specs=[
            pl.BlockSpec((_BM, CP), lambda i: (i, 0)),
            pl.BlockSpec((_BM, CP), lambda i: (i, 0)),
            pl.BlockSpec((2, _BM, 1), lambda i: (0, i, 0)),
        ],
        out_specs=pl.BlockSpec((_BM, CLS), lambda i: (i, 0)),
        out_shape=jax.ShapeDtypeStruct((N, CLS), f32),
    )(q0, q1, norms)


@jax.jit
def kernel(feats, edge_index, W0, W1):
    src = edge_index[0]
    dst = edge_index[1]
    src128 = src.reshape(NW, EPT // EB128, EB128)
    dst128 = dst.reshape(NW, EPT // EB128, EB128)
    src48 = src.reshape(NW, EPT // EB48, EB48)
    dst48 = dst.reshape(NW, EPT // EB48, EB48)
    w1p = jnp.pad(W1, ((0, 0), (0, CP - CLS)))

    zeros128 = jnp.zeros((NP, D_IN), bf16)
    zeros48 = jnp.zeros((NP, CP), bf16)

    degp = _deg_kernel(src, dst)
    degp4 = degp.reshape(NC, 2, NP, 1)
    xn, norms = _tc1(degp4, feats)
    agg1 = _agg128(xn, src128, dst128, zeros128)
    z = _tc2(agg1[0, :N], agg1[1, :N], norms, W0, w1p)
    agg2 = _agg48(z, src48, dst48, zeros48)
    return _tc3(agg2[0, :N], agg2[1, :N], norms)


# speedup vs baseline: 11.1002x; 1.0815x over previous
import functools

import jax
import jax.numpy as jnp
from jax import lax
from jax.experimental import pallas as pl
from jax.experimental.pallas import tpu as pltpu
from jax.experimental.pallas import tpu_sc as plsc

N = 10000
E = 320000
D_IN = 128
HID = 128
CLS = 40
CP = 48

NC, NS, L = 2, 16, 16
NW = NC * NS
NP = 10240
RPT = NP // NS
EPT = E // NW
EB = 80
NCHUNK = EPT // EB

_mesh = plsc.VectorSubcoreMesh(core_axis_name="c", subcore_axis_name="s")
_sc_params = pltpu.CompilerParams(
    needs_layout_passes=False, use_tc_tiling_on_sc=False)
f32 = jnp.float32
bf16 = jnp.bfloat16


@functools.partial(
    pl.kernel,
    out_type=jax.ShapeDtypeStruct((NC, 2, NP), f32),
    mesh=_mesh,
    compiler_params=_sc_params,
    scratch_types=[
        pltpu.VMEM((EPT,), jnp.int32),
        pltpu.VMEM((EPT,), jnp.int32),
        pltpu.VMEM((NP,), f32),
        pltpu.VMEM((NP,), f32),
        pltpu.VMEM((2 * RPT,), f32),
        pltpu.VMEM((2 * RPT,), f32),
        pltpu.VMEM_SHARED((NS, 2, NP), f32),
    ],
)
def _deg_kernel(src_hbm, dst_hbm, out_hbm, si_v, di_v, d0_v, d1_v,
                racc, rtmp, shared):
    c = lax.axis_index("c")
    s = lax.axis_index("s")
    w = c * NS + s
    base = w * EPT
    pltpu.sync_copy(src_hbm.at[pl.ds(base, EPT)], si_v)
    pltpu.sync_copy(dst_hbm.at[pl.ds(base, EPT)], di_v)

    zero16 = jnp.zeros((L,), f32)

    def zero_body(i, _):
        d0_v[pl.ds(i * L, L)] = zero16
        d1_v[pl.ds(i * L, L)] = zero16
        return _

    lax.fori_loop(0, NP // L, zero_body, None)

    ones = jnp.ones((L,), f32)

    def scat_body(i, _):
        plsc.addupdate_scatter(d0_v, [si_v[pl.ds(i * L, L)]], ones)
        plsc.addupdate_scatter(d1_v, [di_v[pl.ds(i * L, L)]], ones)
        return _

    lax.fori_loop(0, EPT // L, scat_body, None)

    pltpu.sync_copy(d0_v, shared.at[s, 0])
    pltpu.sync_copy(d1_v, shared.at[s, 1])
    plsc.subcore_barrier()

    rows = pl.ds(s * RPT, RPT)
    pltpu.sync_copy(shared.at[0, 0, rows], racc.at[pl.ds(0, RPT)])
    pltpu.sync_copy(shared.at[0, 1, rows], racc.at[pl.ds(RPT, RPT)])

    def red_body(j, _):
        pltpu.sync_copy(shared.at[j, 0, rows], rtmp.at[pl.ds(0, RPT)])
        pltpu.sync_copy(shared.at[j, 1, rows], rtmp.at[pl.ds(RPT, RPT)])

        def add_body(v, carry):
            sl = pl.ds(v * L, L)
            racc[sl] = racc[sl] + rtmp[sl]
            return carry

        lax.fori_loop(0, 2 * RPT // L, add_body, None)
        return _

    lax.fori_loop(1, NS, red_body, None)
    pltpu.sync_copy(racc.at[pl.ds(0, RPT)], out_hbm.at[c, 0, rows])
    pltpu.sync_copy(racc.at[pl.ds(RPT, RPT)], out_hbm.at[c, 1, rows])


NBUF = 5


def _make_agg(D, eb, dt):
    nchunk = EPT // eb

    @functools.partial(
        pl.kernel,
        out_type=jax.ShapeDtypeStruct((NC, NP, D), dt),
        mesh=_mesh,
        compiler_params=_sc_params,
        scratch_types=[
            pltpu.VMEM((NBUF, eb), jnp.int32),
            pltpu.VMEM((NBUF, eb), jnp.int32),
            pltpu.VMEM((NBUF, eb, D), dt),
            pltpu.VMEM_SHARED((NP, D), dt),
            pltpu.VMEM_SHARED((NP, D), dt),
        ] + [pltpu.SemaphoreType.DMA] * NBUF,
    )
    def _agg(x_hbm, src_hbm, dst_hbm, zeros_hbm, out_hbm, si2d, di2d, rows_v,
             acc, xtab, *sems):
        c = lax.axis_index("c")
        s = lax.axis_index("s")
        w = c * NS + s

        myrows = pl.ds(s * RPT, RPT)
        pltpu.sync_copy(zeros_hbm, acc.at[myrows])
        pltpu.sync_copy(x_hbm.at[myrows], xtab.at[myrows])
        plsc.subcore_barrier()

        def chunk_body(k, _):
            chunks = pl.ds(k * NBUF, NBUF)
            pltpu.sync_copy(src_hbm.at[w, chunks], si2d)
            pltpu.sync_copy(dst_hbm.at[w, chunks], di2d)
            descs = []
            for b in range(NBUF):
                descs.append(pltpu.async_copy(
                    xtab.at[si2d.at[b]], rows_v.at[b], sems[b]))
            for b in range(NBUF):
                descs[b].wait()
                pltpu.sync_copy(rows_v.at[b], acc.at[di2d.at[b]], add=True)
            return _

        lax.fori_loop(0, nchunk // NBUF, chunk_body, None)
        plsc.subcore_barrier()
        pltpu.sync_copy(acc.at[myrows], out_hbm.at[c].at[myrows])

    return _agg


EB128 = 80
EB48 = 80
_agg128 = _make_agg(D_IN, EB128, bf16)
_agg48 = _make_agg(CP, EB48, bf16)


_BM = 2000
_GRID = N // _BM


def _tc1_body(degp, feats, xn, norms):
    d = degp[0] + degp[1]
    no = lax.rsqrt(jnp.maximum(d[0], 1.0))
    ni = lax.rsqrt(jnp.maximum(d[1], 1.0))
    norms[0] = no
    norms[1] = ni
    xn[...] = (feats[...] * no).astype(bf16)


def _tc1(degp4, feats):
    return pl.pallas_call(
        _tc1_body,
        grid=(_GRID,),
        in_specs=[
            pl.BlockSpec((NC, 2, _BM, 1), lambda i: (0, 0, i, 0)),
            pl.BlockSpec((_BM, D_IN), lambda i: (i, 0)),
        ],
        out_specs=[
            pl.BlockSpec((_BM, D_IN), lambda i: (i, 0)),
            pl.BlockSpec((2, _BM, 1), lambda i: (0, i, 0)),
        ],
        out_shape=[
            jax.ShapeDtypeStruct((N, D_IN), bf16),
            jax.ShapeDtypeStruct((2, N, 1), f32),
        ],
    )(degp4, feats)


def _tc2_body(p0, p1, norms, w0, w1, z):
    a = p0[...].astype(f32) + p1[...].astype(f32)
    ni = norms[1]
    no = norms[0]
    h = jnp.dot(a, w0[...], preferred_element_type=f32) * ni
    h = jnp.maximum(h, 0.0) * no
    z[...] = jnp.dot(h, w1[...], preferred_element_type=f32).astype(bf16)


def _tc2(p0, p1, norms, w0, w1p):
    return pl.pallas_call(
        _tc2_body,
        grid=(_GRID,),
        in_specs=[
            pl.BlockSpec((_BM, D_IN), lambda i: (i, 0)),
            pl.BlockSpec((_BM, D_IN), lambda i: (i, 0)),
            pl.BlockSpec((2, _BM, 1), lambda i: (0, i, 0)),
            pl.BlockSpec((D_IN, HID), lambda i: (0, 0)),
            pl.BlockSpec((HID, CP), lambda i: (0, 0)),
        ],
        out_specs=pl.BlockSpec((_BM, CP), lambda i: (i, 0)),
        out_shape=jax.ShapeDtypeStruct((N, CP), bf16),
    )(p0, p1, norms, w0, w1p)


def _tc3_body(q0, q1, norms, out):
    ni = norms[1]
    out[...] = (q0[...].astype(f32) + q1[...].astype(f32))[:, :CLS] * ni


def _tc3(q0, q1, norms):
    return pl.pallas_call(
        _tc3_body,
        grid=(_GRID,),
        in_specs=[
            pl.BlockSpec((_BM, CP), lambda i: (i, 0)),
            pl.BlockSpec((_BM, CP), lambda i: (i, 0)),
            pl.BlockSpec((2, _BM, 1), lambda i: (0, i, 0)),
        ],
        out_specs=pl.BlockSpec((_BM, CLS), lambda i: (i, 0)),
        out_shape=jax.ShapeDtypeStruct((N, CLS), f32),
    )(q0, q1, norms)


@jax.jit
def kernel(feats, edge_index, W0, W1):
    src = edge_index[0]
    dst = edge_index[1]
    src128 = src.reshape(NW, EPT // EB128, EB128)
    dst128 = dst.reshape(NW, EPT // EB128, EB128)
    src48 = src.reshape(NW, EPT // EB48, EB48)
    dst48 = dst.reshape(NW, EPT // EB48, EB48)
    w1p = jnp.pad(W1, ((0, 0), (0, CP - CLS)))

    zeros128 = jnp.zeros((RPT, D_IN), bf16)
    zeros48 = jnp.zeros((RPT, CP), bf16)

    degp = _deg_kernel(src, dst)
    degp4 = degp.reshape(NC, 2, NP, 1)
    xn, norms = _tc1(degp4, feats)
    agg1 = _agg128(xn, src128, dst128, zeros128)
    z = _tc2(agg1[0, :N], agg1[1, :N], norms, W0, w1p)
    agg2 = _agg48(z, src48, dst48, zeros48)
    return _tc3(agg2[0, :N], agg2[1, :N], norms)


# speedup vs baseline: 11.3216x; 1.0199x over previous
import functools

import jax
import jax.numpy as jnp
from jax import lax
from jax.experimental import pallas as pl
from jax.experimental.pallas import tpu as pltpu
from jax.experimental.pallas import tpu_sc as plsc

N = 10000
E = 320000
D_IN = 128
HID = 128
CLS = 40
CP = 48

NC, NS, L = 2, 16, 16
NW = NC * NS
NP = 10240
RPT = NP // NS
EPT = E // NW
EB = 80
NCHUNK = EPT // EB

_mesh = plsc.VectorSubcoreMesh(core_axis_name="c", subcore_axis_name="s")
_sc_params = pltpu.CompilerParams(
    needs_layout_passes=False, use_tc_tiling_on_sc=False)
f32 = jnp.float32
bf16 = jnp.bfloat16


@functools.partial(
    pl.kernel,
    out_type=jax.ShapeDtypeStruct((NC, 2, NP), f32),
    mesh=_mesh,
    compiler_params=_sc_params,
    scratch_types=[
        pltpu.VMEM((EPT,), jnp.int32),
        pltpu.VMEM((EPT,), jnp.int32),
        pltpu.VMEM((NP,), f32),
        pltpu.VMEM((NP,), f32),
        pltpu.VMEM((2 * RPT,), f32),
        pltpu.VMEM((2 * RPT,), f32),
        pltpu.VMEM_SHARED((NS, 2, NP), f32),
    ],
)
def _deg_kernel(src_hbm, dst_hbm, out_hbm, si_v, di_v, d0_v, d1_v,
                racc, rtmp, shared):
    c = lax.axis_index("c")
    s = lax.axis_index("s")
    w = c * NS + s
    base = w * EPT
    pltpu.sync_copy(src_hbm.at[pl.ds(base, EPT)], si_v)
    pltpu.sync_copy(dst_hbm.at[pl.ds(base, EPT)], di_v)

    zero16 = jnp.zeros((L,), f32)

    def zero_body(i, _):
        d0_v[pl.ds(i * L, L)] = zero16
        d1_v[pl.ds(i * L, L)] = zero16
        return _

    lax.fori_loop(0, NP // L, zero_body, None)

    ones = jnp.ones((L,), f32)

    def scat_body(i, _):
        plsc.addupdate_scatter(d0_v, [si_v[pl.ds(i * L, L)]], ones)
        plsc.addupdate_scatter(d1_v, [di_v[pl.ds(i * L, L)]], ones)
        return _

    lax.fori_loop(0, EPT // L, scat_body, None)

    pltpu.sync_copy(d0_v, shared.at[s, 0])
    pltpu.sync_copy(d1_v, shared.at[s, 1])
    plsc.subcore_barrier()

    rows = pl.ds(s * RPT, RPT)
    pltpu.sync_copy(shared.at[0, 0, rows], racc.at[pl.ds(0, RPT)])
    pltpu.sync_copy(shared.at[0, 1, rows], racc.at[pl.ds(RPT, RPT)])

    def red_body(j, _):
        pltpu.sync_copy(shared.at[j, 0, rows], rtmp.at[pl.ds(0, RPT)])
        pltpu.sync_copy(shared.at[j, 1, rows], rtmp.at[pl.ds(RPT, RPT)])

        def add_body(v, carry):
            sl = pl.ds(v * L, L)
            racc[sl] = racc[sl] + rtmp[sl]
            return carry

        lax.fori_loop(0, 2 * RPT // L, add_body, None)
        return _

    lax.fori_loop(1, NS, red_body, None)
    pltpu.sync_copy(racc.at[pl.ds(0, RPT)], out_hbm.at[c, 0, rows])
    pltpu.sync_copy(racc.at[pl.ds(RPT, RPT)], out_hbm.at[c, 1, rows])


NBUF = 5


def _make_agg(D, eb, dt):
    nchunk = EPT // eb

    @functools.partial(
        pl.kernel,
        out_type=jax.ShapeDtypeStruct((NC, NP, D), dt),
        mesh=_mesh,
        compiler_params=_sc_params,
        scratch_types=[
            pltpu.VMEM((NBUF, eb), jnp.int32),
            pltpu.VMEM((NBUF, eb), jnp.int32),
            pltpu.VMEM((NBUF, eb, D), dt),
            pltpu.VMEM_SHARED((NP, D), dt),
            pltpu.VMEM_SHARED((NP, D), dt),
        ] + [pltpu.SemaphoreType.DMA] * (2 * NBUF),
    )
    def _agg(x_hbm, src_hbm, dst_hbm, zeros_hbm, out_hbm, si2d, di2d, rows_v,
             acc, xtab, *sems):
        c = lax.axis_index("c")
        s = lax.axis_index("s")
        w = c * NS + s

        myrows = pl.ds(s * RPT, RPT)
        pltpu.sync_copy(zeros_hbm, acc.at[myrows])
        pltpu.sync_copy(x_hbm.at[myrows], xtab.at[myrows])
        plsc.subcore_barrier()

        def chunk_body(k, _):
            chunks = pl.ds(k * NBUF, NBUF)
            pltpu.sync_copy(src_hbm.at[w, chunks], si2d)
            pltpu.sync_copy(dst_hbm.at[w, chunks], di2d)
            gd = []
            for b in range(NBUF):
                gd.append(pltpu.async_copy(
                    xtab.at[si2d.at[b]], rows_v.at[b], sems[b]))
            sd = []
            for b in range(NBUF):
                gd[b].wait()
                sd.append(pltpu.async_copy(
                    rows_v.at[b], acc.at[di2d.at[b]], sems[NBUF + b],
                    add=True))
            for b in range(NBUF):
                sd[b].wait()
            return _

        lax.fori_loop(0, nchunk // NBUF, chunk_body, None)
        plsc.subcore_barrier()
        pltpu.sync_copy(acc.at[myrows], out_hbm.at[c].at[myrows])

    return _agg


EB128 = 80
EB48 = 80
_agg128 = _make_agg(D_IN, EB128, bf16)
_agg48 = _make_agg(CP, EB48, bf16)


_BM = 2000
_GRID = N // _BM


def _tc1_body(degp, feats, xn, norms):
    d = degp[0] + degp[1]
    no = lax.rsqrt(jnp.maximum(d[0], 1.0))
    ni = lax.rsqrt(jnp.maximum(d[1], 1.0))
    norms[0] = no
    norms[1] = ni
    xn[...] = (feats[...] * no).astype(bf16)


def _tc1(degp4, feats):
    return pl.pallas_call(
        _tc1_body,
        grid=(_GRID,),
        in_specs=[
            pl.BlockSpec((NC, 2, _BM, 1), lambda i: (0, 0, i, 0)),
            pl.BlockSpec((_BM, D_IN), lambda i: (i, 0)),
        ],
        out_specs=[
            pl.BlockSpec((_BM, D_IN), lambda i: (i, 0)),
            pl.BlockSpec((2, _BM, 1), lambda i: (0, i, 0)),
        ],
        out_shape=[
            jax.ShapeDtypeStruct((N, D_IN), bf16),
            jax.ShapeDtypeStruct((2, N, 1), f32),
        ],
    )(degp4, feats)


def _tc2_body(p0, p1, norms, w0, w1, z):
    a = p0[...].astype(f32) + p1[...].astype(f32)
    ni = norms[1]
    no = norms[0]
    h = jnp.dot(a, w0[...], preferred_element_type=f32) * ni
    h = jnp.maximum(h, 0.0) * no
    z[...] = jnp.dot(h, w1[...], preferred_element_type=f32).astype(bf16)


def _tc2(p0, p1, norms, w0, w1p):
    return pl.pallas_call(
        _tc2_body,
        grid=(_GRID,),
        in_specs=[
            pl.BlockSpec((_BM, D_IN), lambda i: (i, 0)),
            pl.BlockSpec((_BM, D_IN), lambda i: (i, 0)),
            pl.BlockSpec((2, _BM, 1), lambda i: (0, i, 0)),
            pl.BlockSpec((D_IN, HID), lambda i: (0, 0)),
            pl.BlockSpec((HID, CP), lambda i: (0, 0)),
        ],
        out_specs=pl.BlockSpec((_BM, CP), lambda i: (i, 0)),
        out_shape=jax.ShapeDtypeStruct((N, CP), bf16),
    )(p0, p1, norms, w0, w1p)


def _tc3_body(q0, q1, norms, out):
    ni = norms[1]
    out[...] = (q0[...].astype(f32) + q1[...].astype(f32))[:, :CLS] * ni


def _tc3(q0, q1, norms):
    return pl.pallas_call(
        _tc3_body,
        grid=(_GRID,),
        in_specs=[
            pl.BlockSpec((_BM, CP), lambda i: (i, 0)),
            pl.BlockSpec((_BM, CP), lambda i: (i, 0)),
            pl.BlockSpec((2, _BM, 1), lambda i: (0, i, 0)),
        ],
        out_specs=pl.BlockSpec((_BM, CLS), lambda i: (i, 0)),
        out_shape=jax.ShapeDtypeStruct((N, CLS), f32),
    )(q0, q1, norms)


@jax.jit
def kernel(feats, edge_index, W0, W1):
    src = edge_index[0]
    dst = edge_index[1]
    src128 = src.reshape(NW, EPT // EB128, EB128)
    dst128 = dst.reshape(NW, EPT // EB128, EB128)
    src48 = src.reshape(NW, EPT // EB48, EB48)
    dst48 = dst.reshape(NW, EPT // EB48, EB48)
    w1p = jnp.pad(W1, ((0, 0), (0, CP - CLS)))

    zeros128 = jnp.zeros((RPT, D_IN), bf16)
    zeros48 = jnp.zeros((RPT, CP), bf16)

    degp = _deg_kernel(src, dst)
    degp4 = degp.reshape(NC, 2, NP, 1)
    xn, norms = _tc1(degp4, feats)
    agg1 = _agg128(xn, src128, dst128, zeros128)
    z = _tc2(agg1[0, :N], agg1[1, :N], norms, W0, w1p)
    agg2 = _agg48(z, src48, dst48, zeros48)
    return _tc3(agg2[0, :N], agg2[1, :N], norms)


# speedup vs baseline: 11.4189x; 1.0086x over previous
import functools

import jax
import jax.numpy as jnp
from jax import lax
from jax.experimental import pallas as pl
from jax.experimental.pallas import tpu as pltpu
from jax.experimental.pallas import tpu_sc as plsc

N = 10000
E = 320000
D_IN = 128
HID = 128
CLS = 40
CP = 48

NC, NS, L = 2, 16, 16
NW = NC * NS
NP = 10240
RPT = NP // NS
EPT = E // NW
EB = 80
NCHUNK = EPT // EB

_mesh = plsc.VectorSubcoreMesh(core_axis_name="c", subcore_axis_name="s")
_sc_params = pltpu.CompilerParams(
    needs_layout_passes=False, use_tc_tiling_on_sc=False)
f32 = jnp.float32
bf16 = jnp.bfloat16


@functools.partial(
    pl.kernel,
    out_type=jax.ShapeDtypeStruct((NC, 2, NP), f32),
    mesh=_mesh,
    compiler_params=_sc_params,
    scratch_types=[
        pltpu.VMEM((EPT,), jnp.int32),
        pltpu.VMEM((EPT,), jnp.int32),
        pltpu.VMEM((NP,), f32),
        pltpu.VMEM((NP,), f32),
        pltpu.VMEM((2 * RPT,), f32),
        pltpu.VMEM((2 * RPT,), f32),
        pltpu.VMEM_SHARED((NS, 2, NP), f32),
    ],
)
def _deg_kernel(src_hbm, dst_hbm, out_hbm, si_v, di_v, d0_v, d1_v,
                racc, rtmp, shared):
    c = lax.axis_index("c")
    s = lax.axis_index("s")
    w = c * NS + s
    base = w * EPT
    pltpu.sync_copy(src_hbm.at[pl.ds(base, EPT)], si_v)
    pltpu.sync_copy(dst_hbm.at[pl.ds(base, EPT)], di_v)

    zero16 = jnp.zeros((L,), f32)

    def zero_body(i, _):
        d0_v[pl.ds(i * L, L)] = zero16
        d1_v[pl.ds(i * L, L)] = zero16
        return _

    lax.fori_loop(0, NP // L, zero_body, None)

    ones = jnp.ones((L,), f32)

    def scat_body(i, _):
        plsc.addupdate_scatter(d0_v, [si_v[pl.ds(i * L, L)]], ones)
        plsc.addupdate_scatter(d1_v, [di_v[pl.ds(i * L, L)]], ones)
        return _

    lax.fori_loop(0, EPT // L, scat_body, None)

    pltpu.sync_copy(d0_v, shared.at[s, 0])
    pltpu.sync_copy(d1_v, shared.at[s, 1])
    plsc.subcore_barrier()

    rows = pl.ds(s * RPT, RPT)
    pltpu.sync_copy(shared.at[0, 0, rows], racc.at[pl.ds(0, RPT)])
    pltpu.sync_copy(shared.at[0, 1, rows], racc.at[pl.ds(RPT, RPT)])

    def red_body(j, _):
        pltpu.sync_copy(shared.at[j, 0, rows], rtmp.at[pl.ds(0, RPT)])
        pltpu.sync_copy(shared.at[j, 1, rows], rtmp.at[pl.ds(RPT, RPT)])

        def add_body(v, carry):
            sl = pl.ds(v * L, L)
            racc[sl] = racc[sl] + rtmp[sl]
            return carry

        lax.fori_loop(0, 2 * RPT // L, add_body, None)
        return _

    lax.fori_loop(1, NS, red_body, None)
    pltpu.sync_copy(racc.at[pl.ds(0, RPT)], out_hbm.at[c, 0, rows])
    pltpu.sync_copy(racc.at[pl.ds(RPT, RPT)], out_hbm.at[c, 1, rows])


NBUF = 5


def _make_agg(D, eb, dt):
    nchunk = EPT // eb

    @functools.partial(
        pl.kernel,
        out_type=jax.ShapeDtypeStruct((NC, NP, D), dt),
        mesh=_mesh,
        compiler_params=_sc_params,
        scratch_types=[
            pltpu.VMEM((NBUF, eb), jnp.int32),
            pltpu.VMEM((NBUF, eb), jnp.int32),
            pltpu.VMEM((NBUF, eb, D), dt),
            pltpu.VMEM_SHARED((NP, D), dt),
            pltpu.VMEM_SHARED((NP, D), dt),
        ] + [pltpu.SemaphoreType.DMA] * (2 * NBUF),
    )
    def _agg(x_hbm, src_hbm, dst_hbm, zeros_hbm, out_hbm, si2d, di2d, rows_v,
             acc, xtab, *sems):
        c = lax.axis_index("c")
        s = lax.axis_index("s")
        w = c * NS + s

        myrows = pl.ds(s * RPT, RPT)
        pltpu.sync_copy(zeros_hbm, acc.at[myrows])
        pltpu.sync_copy(x_hbm.at[myrows], xtab.at[myrows])
        plsc.subcore_barrier()

        def chunk_body(k, _):
            chunks = pl.ds(k * NBUF, NBUF)
            pltpu.sync_copy(src_hbm.at[w, chunks], si2d)
            pltpu.sync_copy(dst_hbm.at[w, chunks], di2d)
            gd = []
            for b in range(NBUF):
                gd.append(pltpu.async_copy(
                    xtab.at[si2d.at[b]], rows_v.at[b], sems[b]))
            sd = []
            for b in range(NBUF):
                gd[b].wait()
                sd.append(pltpu.async_copy(
                    rows_v.at[b], acc.at[di2d.at[b]], sems[NBUF + b],
                    add=True))
            for b in range(NBUF):
                sd[b].wait()
            return _

        lax.fori_loop(0, nchunk // NBUF, chunk_body, None)
        plsc.subcore_barrier()
        pltpu.sync_copy(acc.at[myrows], out_hbm.at[c].at[myrows])

    return _agg


EB128 = 80
EB48 = 80
_agg128 = _make_agg(D_IN, EB128, bf16)
_agg48 = _make_agg(CP, EB48, bf16)


_BM = 2000
_GRID = N // _BM


def _tc1_body(degp, feats, xn, norms):
    d = degp[0] + degp[1]
    no = lax.rsqrt(jnp.maximum(d[0], 1.0))
    ni = lax.rsqrt(jnp.maximum(d[1], 1.0))
    norms[0] = no
    norms[1] = ni
    xn[...] = (feats[...] * no).astype(bf16)


def _tc1(degp4, feats):
    return pl.pallas_call(
        _tc1_body,
        grid=(_GRID,),
        in_specs=[
            pl.BlockSpec((NC, 2, _BM, 1), lambda i: (0, 0, i, 0)),
            pl.BlockSpec((_BM, D_IN), lambda i: (i, 0)),
        ],
        out_specs=[
            pl.BlockSpec((_BM, D_IN), lambda i: (i, 0)),
            pl.BlockSpec((2, _BM, 1), lambda i: (0, i, 0)),
        ],
        out_shape=[
            jax.ShapeDtypeStruct((NP, D_IN), bf16),
            jax.ShapeDtypeStruct((2, N, 1), f32),
        ],
    )(degp4, feats)


def _tc2_body(p, norms, w0, w1, z):
    a = p[0].astype(f32) + p[1].astype(f32)
    ni = norms[1]
    no = norms[0]
    h = jnp.dot(a, w0[...], preferred_element_type=f32) * ni
    h = jnp.maximum(h, 0.0) * no
    z[...] = jnp.dot(h, w1[...], preferred_element_type=f32).astype(bf16)


def _tc2(p, norms, w0, w1p):
    return pl.pallas_call(
        _tc2_body,
        grid=(_GRID,),
        in_specs=[
            pl.BlockSpec((NC, _BM, D_IN), lambda i: (0, i, 0)),
            pl.BlockSpec((2, _BM, 1), lambda i: (0, i, 0)),
            pl.BlockSpec((D_IN, HID), lambda i: (0, 0)),
            pl.BlockSpec((HID, CP), lambda i: (0, 0)),
        ],
        out_specs=pl.BlockSpec((_BM, CP), lambda i: (i, 0)),
        out_shape=jax.ShapeDtypeStruct((NP, CP), bf16),
    )(p, norms, w0, w1p)


def _tc3_body(q, norms, out):
    ni = norms[1]
    out[...] = (q[0].astype(f32) + q[1].astype(f32))[:, :CLS] * ni


def _tc3(q, norms):
    return pl.pallas_call(
        _tc3_body,
        grid=(_GRID,),
        in_specs=[
            pl.BlockSpec((NC, _BM, CP), lambda i: (0, i, 0)),
            pl.BlockSpec((2, _BM, 1), lambda i: (0, i, 0)),
        ],
        out_specs=pl.BlockSpec((_BM, CLS), lambda i: (i, 0)),
        out_shape=jax.ShapeDtypeStruct((N, CLS), f32),
    )(q, norms)


@jax.jit
def kernel(feats, edge_index, W0, W1):
    src = edge_index[0]
    dst = edge_index[1]
    src128 = src.reshape(NW, EPT // EB128, EB128)
    dst128 = dst.reshape(NW, EPT // EB128, EB128)
    src48 = src.reshape(NW, EPT // EB48, EB48)
    dst48 = dst.reshape(NW, EPT // EB48, EB48)
    w1p = jnp.pad(W1, ((0, 0), (0, CP - CLS)))

    zeros128 = jnp.zeros((RPT, D_IN), bf16)
    zeros48 = jnp.zeros((RPT, CP), bf16)

    degp = _deg_kernel(src, dst)
    degp4 = degp.reshape(NC, 2, NP, 1)
    xn, norms = _tc1(degp4, feats)
    agg1 = _agg128(xn, src128, dst128, zeros128)
    z = _tc2(agg1, norms, W0, w1p)
    agg2 = _agg48(z, src48, dst48, zeros48)
    return _tc3(agg2, norms)
